# Initial kernel scaffold; baseline (speedup 1.0000x reference)
import math

import jax
import jax.numpy as jnp
import numpy as np
from jax import lax
from jax.experimental import pallas as pl
from jax.experimental.pallas import tpu as pltpu
from jax.experimental.pallas import tpu_sc as plsc

C_Z = 128
NUM_HEADS = 12
HEAD_DIM = C_Z // NUM_HEADS
EDGE_DIM = C_Z // 4
N_NODES = 10000
N_EDGES = 320000
NUM_GAUSS = EDGE_DIM
STOP = 15.0
HD = NUM_HEADS * HEAD_DIM
PADW = 128
PDIM = 16

SC_CORES = 2
SC_TILES = 16
N_WORKERS = SC_CORES * SC_TILES
EPW = N_EDGES // N_WORKERS
CHUNK = 400
NCHUNK = EPW // CHUNK
NPT = N_NODES // SC_TILES

_RBF_STEP = STOP / (NUM_GAUSS - 1)
_RBF_COEFF = -0.5 / _RBF_STEP**2
_INV_SQRT_D = 1.0 / math.sqrt(float(HEAD_DIM))

NBLK = 1000
EBLK = 2000


def _prep_body(h_ref, wq_ref, bq_ref, q_ref):
    q_ref[...] = (
        jnp.dot(h_ref[...], wq_ref[...], preferred_element_type=jnp.float32)
        + bq_ref[...]
    )


def _edge_body(qr_ref, hc_ref, pr_ref, pc_ref, wkh_ref, wke_ref, bk_ref,
               wvh_ref, wve_ref, bv_ref, we_ref, be_ref, shead_ref, ehead_ref,
               ev_ref, e16_ref):
    d = pr_ref[...] - pc_ref[...]
    d2 = jnp.sum(d * d, axis=1, keepdims=True)
    dist = jnp.sqrt(d2 + 1e-12)
    offs = (lax.broadcasted_iota(jnp.float32, (1, NUM_GAUSS), 1) * _RBF_STEP)
    rbf = jnp.exp(_RBF_COEFF * (dist - offs) ** 2)
    ea = (jnp.dot(rbf, we_ref[...], preferred_element_type=jnp.float32)
          + be_ref[...])
    hc = hc_ref[...]
    k = (jnp.dot(hc, wkh_ref[...], preferred_element_type=jnp.float32)
         + jnp.dot(ea, wke_ref[...], preferred_element_type=jnp.float32)
         + bk_ref[...])
    v = (jnp.dot(hc, wvh_ref[...], preferred_element_type=jnp.float32)
         + jnp.dot(ea, wve_ref[...], preferred_element_type=jnp.float32)
         + bv_ref[...])
    qk = qr_ref[...] * k
    alpha = jnp.dot(qk, shead_ref[...], preferred_element_type=jnp.float32)
    e = jnp.exp(alpha)
    ev = v * jnp.dot(e, ehead_ref[...], preferred_element_type=jnp.float32)
    ev_ref[...] = ev
    e16_ref[...] = e


def _final_body(h_ref, n0_ref, n1_ref, d0_ref, d1_ref, wo_ref, bo_ref,
                ehead_ref, out_ref):
    num = n0_ref[...] + n1_ref[...]
    den = d0_ref[...] + d1_ref[...]
    dexp = jnp.dot(den, ehead_ref[...], preferred_element_type=jnp.float32)
    agg = num / (dexp + 1e-16)
    out_ref[...] = (
        h_ref[...]
        + jnp.dot(agg, wo_ref[...], preferred_element_type=jnp.float32)
        + bo_ref[...]
    )


def _sc_gather_body(q_hbm, h_hbm, p_hbm, row_hbm, col_hbm,
                    qr_hbm, hc_hbm, pr_hbm, pc_hbm,
                    ri_v, ci_v, q_v, h_v, pr_v, pc_v, sem):
    wid = lax.axis_index("s") * SC_CORES + lax.axis_index("c")
    base = wid * EPW

    @pl.loop(0, NCHUNK)
    def _(j):
        off = base + j * CHUNK
        pltpu.sync_copy(row_hbm.at[pl.ds(off, CHUNK)], ri_v)
        pltpu.sync_copy(col_hbm.at[pl.ds(off, CHUNK)], ci_v)
        c1 = pltpu.async_copy(q_hbm.at[ri_v], q_v, sem)
        c2 = pltpu.async_copy(h_hbm.at[ci_v], h_v, sem)
        c3 = pltpu.async_copy(p_hbm.at[ri_v], pr_v, sem)
        c4 = pltpu.async_copy(p_hbm.at[ci_v], pc_v, sem)
        c1.wait()
        c2.wait()
        c3.wait()
        c4.wait()
        pltpu.sync_copy(q_v, qr_hbm.at[pl.ds(off, CHUNK)])
        pltpu.sync_copy(h_v, hc_hbm.at[pl.ds(off, CHUNK)])
        pltpu.sync_copy(pr_v, pr_hbm.at[pl.ds(off, CHUNK)])
        pltpu.sync_copy(pc_v, pc_hbm.at[pl.ds(off, CHUNK)])


def _sc_scatter_body(ev_hbm, e16_hbm, row_hbm, zn_hbm, zd_hbm,
                     num_hbm, den_hbm,
                     ri_v, ev_v, e_v, accn_s, accd_s, sem):
    cid = lax.axis_index("c")
    sid = lax.axis_index("s")
    pltpu.sync_copy(zn_hbm.at[pl.ds(sid * NPT, NPT)],
                    accn_s.at[pl.ds(sid * NPT, NPT)])
    pltpu.sync_copy(zd_hbm.at[pl.ds(sid * NPT, NPT)],
                    accd_s.at[pl.ds(sid * NPT, NPT)])
    plsc.subcore_barrier()

    wid = sid * SC_CORES + cid
    base = wid * EPW

    @pl.loop(0, NCHUNK)
    def _(j):
        off = base + j * CHUNK
        pltpu.sync_copy(row_hbm.at[pl.ds(off, CHUNK)], ri_v)
        c1 = pltpu.async_copy(ev_hbm.at[pl.ds(off, CHUNK)], ev_v, sem)
        c2 = pltpu.async_copy(e16_hbm.at[pl.ds(off, CHUNK)], e_v, sem)
        c1.wait()
        c2.wait()
        pltpu.sync_copy(ev_v, accn_s.at[ri_v], add=True)
        pltpu.sync_copy(e_v, accd_s.at[ri_v], add=True)

    plsc.subcore_barrier()
    pltpu.sync_copy(accn_s.at[pl.ds(sid * NPT, NPT)],
                    num_hbm.at[cid, pl.ds(sid * NPT, NPT)])
    pltpu.sync_copy(accd_s.at[pl.ds(sid * NPT, NPT)],
                    den_hbm.at[cid, pl.ds(sid * NPT, NPT)])


def kernel(pos, h, edge_index, We, be, Wq, bq, Wk, bk, Wv, bv, Wo, bo):
    f32 = jnp.float32
    row = edge_index[0].astype(jnp.int32)
    col = edge_index[1].astype(jnp.int32)

    scale = _INV_SQRT_D
    wq_p = jnp.zeros((C_Z, PADW), f32).at[:, :HD].set(Wq * scale)
    bq_p = jnp.zeros((1, PADW), f32).at[0, :HD].set(bq * scale)
    wkh_p = jnp.zeros((C_Z, PADW), f32).at[:, :HD].set(Wk[:C_Z])
    wke_p = jnp.zeros((EDGE_DIM, PADW), f32).at[:, :HD].set(Wk[C_Z:])
    bk_p = jnp.zeros((1, PADW), f32).at[0, :HD].set(bk)
    wvh_p = jnp.zeros((C_Z, PADW), f32).at[:, :HD].set(Wv[:C_Z])
    wve_p = jnp.zeros((EDGE_DIM, PADW), f32).at[:, :HD].set(Wv[C_Z:])
    bv_p = jnp.zeros((1, PADW), f32).at[0, :HD].set(bv)
    wo_p = jnp.zeros((PADW, C_Z), f32).at[:HD, :].set(Wo)
    bo_p = bo.reshape(1, C_Z)
    be_p = be.reshape(1, EDGE_DIM)
    sh = np.zeros((PADW, PDIM), np.float32)
    for dd in range(HD):
        sh[dd, dd // HEAD_DIM] = 1.0
    shead = jnp.asarray(sh)
    ehead = jnp.asarray(np.ascontiguousarray(sh.T))
    p_pad = jnp.zeros((N_NODES, PDIM), f32).at[:, :3].set(pos)

    nblocks = N_NODES // NBLK
    q = pl.pallas_call(
        _prep_body,
        grid=(nblocks,),
        in_specs=[
            pl.BlockSpec((NBLK, C_Z), lambda i: (i, 0)),
            pl.BlockSpec((C_Z, PADW), lambda i: (0, 0)),
            pl.BlockSpec((1, PADW), lambda i: (0, 0)),
        ],
        out_specs=pl.BlockSpec((NBLK, PADW), lambda i: (i, 0)),
        out_shape=jax.ShapeDtypeStruct((N_NODES, PADW), f32),
    )(h, wq_p, bq_p)

    mesh = plsc.VectorSubcoreMesh(core_axis_name="c", subcore_axis_name="s")
    gather = pl.kernel(
        _sc_gather_body,
        out_type=[
            jax.ShapeDtypeStruct((N_EDGES, PADW), f32),
            jax.ShapeDtypeStruct((N_EDGES, PADW), f32),
            jax.ShapeDtypeStruct((N_EDGES, PDIM), f32),
            jax.ShapeDtypeStruct((N_EDGES, PDIM), f32),
        ],
        mesh=mesh,
        scratch_types=[
            pltpu.VMEM((CHUNK,), jnp.int32),
            pltpu.VMEM((CHUNK,), jnp.int32),
            pltpu.VMEM((CHUNK, PADW), f32),
            pltpu.VMEM((CHUNK, PADW), f32),
            pltpu.VMEM((CHUNK, PDIM), f32),
            pltpu.VMEM((CHUNK, PDIM), f32),
            pltpu.SemaphoreType.DMA,
        ],
    )
    qr, hc, pr, pc = gather(q, h, p_pad, row, col)

    eblocks = N_EDGES // EBLK
    full = lambda r, c: pl.BlockSpec((r, c), lambda i: (0, 0))
    ev, e16 = pl.pallas_call(
        _edge_body,
        grid=(eblocks,),
        in_specs=[
            pl.BlockSpec((EBLK, PADW), lambda i: (i, 0)),
            pl.BlockSpec((EBLK, PADW), lambda i: (i, 0)),
            pl.BlockSpec((EBLK, PDIM), lambda i: (i, 0)),
            pl.BlockSpec((EBLK, PDIM), lambda i: (i, 0)),
            full(C_Z, PADW), full(EDGE_DIM, PADW), full(1, PADW),
            full(C_Z, PADW), full(EDGE_DIM, PADW), full(1, PADW),
            full(EDGE_DIM, EDGE_DIM), full(1, EDGE_DIM),
            full(PADW, PDIM), full(PDIM, PADW),
        ],
        out_specs=[
            pl.BlockSpec((EBLK, PADW), lambda i: (i, 0)),
            pl.BlockSpec((EBLK, PDIM), lambda i: (i, 0)),
        ],
        out_shape=[
            jax.ShapeDtypeStruct((N_EDGES, PADW), f32),
            jax.ShapeDtypeStruct((N_EDGES, PDIM), f32),
        ],
    )(qr, hc, pr, pc, wkh_p, wke_p, bk_p, wvh_p, wve_p, bv_p,
      We, be_p, shead, ehead)

    zn = jnp.zeros((N_NODES, PADW), f32)
    zd = jnp.zeros((N_NODES, PDIM), f32)
    scatter = pl.kernel(
        _sc_scatter_body,
        out_type=[
            jax.ShapeDtypeStruct((SC_CORES, N_NODES, PADW), f32),
            jax.ShapeDtypeStruct((SC_CORES, N_NODES, PDIM), f32),
        ],
        mesh=mesh,
        scratch_types=[
            pltpu.VMEM((CHUNK,), jnp.int32),
            pltpu.VMEM((CHUNK, PADW), f32),
            pltpu.VMEM((CHUNK, PDIM), f32),
            pltpu.VMEM_SHARED((N_NODES, PADW), f32),
            pltpu.VMEM_SHARED((N_NODES, PDIM), f32),
            pltpu.SemaphoreType.DMA,
        ],
    )
    num2, den2 = scatter(ev, e16, row, zn, zd)

    out = pl.pallas_call(
        _final_body,
        grid=(nblocks,),
        in_specs=[
            pl.BlockSpec((NBLK, C_Z), lambda i: (i, 0)),
            pl.BlockSpec((NBLK, PADW), lambda i: (i, 0)),
            pl.BlockSpec((NBLK, PADW), lambda i: (i, 0)),
            pl.BlockSpec((NBLK, PDIM), lambda i: (i, 0)),
            pl.BlockSpec((NBLK, PDIM), lambda i: (i, 0)),
            full(PADW, C_Z), full(1, C_Z), full(PDIM, PADW),
        ],
        out_specs=pl.BlockSpec((NBLK, C_Z), lambda i: (i, 0)),
        out_shape=jax.ShapeDtypeStruct((N_NODES, C_Z), f32),
    )(h, num2[0], num2[1], den2[0], den2[1], wo_p, bo_p, ehead)
    return out


# speedup vs baseline: 40.1756x; 40.1756x over previous
import math

import jax
import jax.numpy as jnp
import numpy as np
from jax import lax
from jax.experimental import pallas as pl
from jax.experimental.pallas import tpu as pltpu
from jax.experimental.pallas import tpu_sc as plsc

C_Z = 128
NUM_HEADS = 12
HEAD_DIM = C_Z // NUM_HEADS
EDGE_DIM = C_Z // 4
N_NODES = 10000
N_EDGES = 320000
NUM_GAUSS = EDGE_DIM
STOP = 15.0
HD = NUM_HEADS * HEAD_DIM
PADW = 128
BW = 2 * PADW
PDIM = 16

SC_CORES = 2
SC_TILES = 16
N_WORKERS = SC_CORES * SC_TILES
EPW = N_EDGES // N_WORKERS
CG = 200
NCG = EPW // CG
EPT = N_EDGES // SC_TILES
CS = 200
NCS = EPT // CS
NACC = 10240
NPT = NACC // SC_TILES

_RBF_STEP = STOP / (NUM_GAUSS - 1)
_RBF_COEFF = -0.5 / _RBF_STEP**2
_INV_SQRT_D = 1.0 / math.sqrt(float(HEAD_DIM))

NBLK = 1000
EBLK = 2000


def _prep_body(h_ref, p_ref, wq_ref, bq_ref, wb_ref, bb_ref, sa_ref, sb_ref,
               a_ref, b_ref):
    h = h_ref[...]
    p16 = p_ref[...]
    a_ref[...] = (
        jnp.dot(h, wq_ref[...], preferred_element_type=jnp.float32)
        + bq_ref[...]
        + jnp.dot(p16, sa_ref[...], preferred_element_type=jnp.float32)
    )
    b_ref[...] = (
        jnp.dot(h, wb_ref[...], preferred_element_type=jnp.float32)
        + bb_ref[...]
        + jnp.dot(p16, sb_ref[...], preferred_element_type=jnp.float32)
    )


def _edge_body(ar_ref, bc_ref, wke_ref, wve_ref, we_ref, be_ref, shead_ref,
               ehead_ref, f0_ref, f1_ref, pm_ref, o0_ref, o1_ref):
    ar = ar_ref[...]
    bc1 = bc_ref[:, :PADW]
    bc2 = bc_ref[:, PADW:]
    dd = (ar - bc1) * pm_ref[...]
    d2 = jnp.sum(dd * dd, axis=1, keepdims=True)
    dist = jnp.sqrt(d2 + 1e-12)
    offs = (lax.broadcasted_iota(jnp.int32, (1, NUM_GAUSS), 1)
            .astype(jnp.float32) * _RBF_STEP)
    rbf = jnp.exp(_RBF_COEFF * (dist - offs) ** 2)
    ea = (jnp.dot(rbf, we_ref[...], preferred_element_type=jnp.float32)
          + be_ref[...])
    k = bc1 + jnp.dot(ea, wke_ref[...], preferred_element_type=jnp.float32)
    v = bc2 + jnp.dot(ea, wve_ref[...], preferred_element_type=jnp.float32)
    alpha = jnp.dot(ar * k, shead_ref[...],
                    preferred_element_type=jnp.float32)
    e = jnp.exp(alpha)
    ev = v * jnp.dot(e, ehead_ref[...], preferred_element_type=jnp.float32)
    o0_ref[...] = ev + jnp.dot(e, f0_ref[...],
                               preferred_element_type=jnp.float32)
    o1_ref[...] = jnp.dot(e, f1_ref[...], preferred_element_type=jnp.float32)


def _final_body(h_ref, a0_ref, a1_ref, wo_ref, bo_ref, t0_ref, t1_ref,
                ehead_ref, em_ref, out_ref):
    a0 = a0_ref[...]
    a1 = a1_ref[...]
    num = a0 * em_ref[...]
    den16 = (jnp.dot(a0, t0_ref[...], preferred_element_type=jnp.float32)
             + jnp.dot(a1, t1_ref[...], preferred_element_type=jnp.float32))
    dexp = jnp.dot(den16, ehead_ref[...], preferred_element_type=jnp.float32)
    agg = num / (dexp + 1e-16)
    out_ref[...] = (
        h_ref[...]
        + jnp.dot(agg, wo_ref[...], preferred_element_type=jnp.float32)
        + bo_ref[...]
    )


def _sc_gather_body(a_hbm, b_hbm, row_hbm, col_hbm, ar_hbm, bc_hbm,
                    ri_v, ci_v, a_v, b_v, sem):
    wid = lax.axis_index("s") * SC_CORES + lax.axis_index("c")
    base = wid * EPW

    @pl.loop(0, NCG)
    def _(j):
        off = base + j * CG
        pltpu.sync_copy(row_hbm.at[pl.ds(off, CG)], ri_v)
        pltpu.sync_copy(col_hbm.at[pl.ds(off, CG)], ci_v)
        c1 = pltpu.async_copy(a_hbm.at[ri_v], a_v, sem)
        c2 = pltpu.async_copy(b_hbm.at[ci_v], b_v, sem)
        c1.wait()
        c2.wait()
        pltpu.sync_copy(a_v, ar_hbm.at[pl.ds(off, CG)])
        pltpu.sync_copy(b_v, bc_hbm.at[pl.ds(off, CG)])


def _sc_scatter_body(o0_hbm, o1_hbm, row_hbm, zn_hbm, acc_hbm,
                     ri_v, val_v, acc_s, sem):
    cid = lax.axis_index("c")
    sid = lax.axis_index("s")
    pltpu.sync_copy(zn_hbm.at[pl.ds(sid * NPT, NPT)],
                    acc_s.at[pl.ds(sid * NPT, NPT)])
    plsc.subcore_barrier()

    base = sid * EPT

    def _run(src_hbm):
        @pl.loop(0, NCS)
        def _(j):
            off = base + j * CS
            pltpu.sync_copy(row_hbm.at[pl.ds(off, CS)], ri_v)
            pltpu.sync_copy(src_hbm.at[pl.ds(off, CS)], val_v)
            pltpu.sync_copy(val_v, acc_s.at[ri_v], add=True)

    @pl.when(cid == 0)
    def _():
        _run(o0_hbm)

    @pl.when(cid == 1)
    def _():
        _run(o1_hbm)

    plsc.subcore_barrier()
    pltpu.sync_copy(acc_s.at[pl.ds(sid * NPT, NPT)],
                    acc_hbm.at[cid, pl.ds(sid * NPT, NPT)])


def _build_constants(We, be, Wq, bq, Wk, bk, Wv, bv, Wo, bo):
    f32 = jnp.float32
    s = _INV_SQRT_D
    wq_p = jnp.zeros((C_Z, PADW), f32).at[:, :HD].set(Wq * s)
    bq_p = jnp.zeros((1, PADW), f32).at[0, :HD].set(bq * s)
    wb_p = (jnp.zeros((C_Z, BW), f32)
            .at[:, :HD].set(Wk[:C_Z])
            .at[:, PADW:PADW + HD].set(Wv[:C_Z]))
    bb_p = (jnp.zeros((1, BW), f32)
            .at[0, :HD].set(bk)
            .at[0, PADW:PADW + HD].set(bv))
    wke_p = jnp.zeros((EDGE_DIM, PADW), f32).at[:, :HD].set(Wk[C_Z:])
    wve_p = jnp.zeros((EDGE_DIM, PADW), f32).at[:, :HD].set(Wv[C_Z:])
    wo_p = jnp.zeros((PADW, C_Z), f32).at[:HD, :].set(Wo)
    bo_p = bo.reshape(1, C_Z)
    be_p = be.reshape(1, EDGE_DIM)

    sh = np.zeros((PADW, PDIM), np.float32)
    for dd in range(HD):
        sh[dd, dd // HEAD_DIM] = 1.0
    f0 = np.zeros((PDIM, PADW), np.float32)
    for j in range(8):
        f0[j, HD + j] = 1.0
    f1 = np.zeros((PDIM, PADW), np.float32)
    for j in range(8, NUM_HEADS):
        f1[j, j - 8] = 1.0
    sa = np.zeros((PDIM, PADW), np.float32)
    sb = np.zeros((PDIM, BW), np.float32)
    for j in range(3):
        sa[j, HD + j] = 1.0
        sb[j, HD + j] = 1.0
    pm = np.zeros((1, PADW), np.float32)
    pm[0, HD:HD + 3] = 1.0
    em = np.zeros((1, PADW), np.float32)
    em[0, :HD] = 1.0

    return dict(
        wq_p=wq_p, bq_p=bq_p, wb_p=wb_p, bb_p=bb_p, wke_p=wke_p, wve_p=wve_p,
        wo_p=wo_p, bo_p=bo_p, be_p=be_p, We=We,
        shead=jnp.asarray(sh), ehead=jnp.asarray(np.ascontiguousarray(sh.T)),
        f0=jnp.asarray(f0), f1=jnp.asarray(f1),
        t0=jnp.asarray(np.ascontiguousarray(f0.T)),
        t1=jnp.asarray(np.ascontiguousarray(f1.T)),
        sa=jnp.asarray(sa), sb=jnp.asarray(sb),
        pm=jnp.asarray(pm), em=jnp.asarray(em),
    )


_FULL = lambda r, c: pl.BlockSpec((r, c), lambda i: (0, 0))


def _phase1(h, p_pad, c):
    nblocks = N_NODES // NBLK
    return pl.pallas_call(
        _prep_body,
        grid=(nblocks,),
        in_specs=[
            pl.BlockSpec((NBLK, C_Z), lambda i: (i, 0)),
            pl.BlockSpec((NBLK, PDIM), lambda i: (i, 0)),
            _FULL(C_Z, PADW), _FULL(1, PADW),
            _FULL(C_Z, BW), _FULL(1, BW),
            _FULL(PDIM, PADW), _FULL(PDIM, BW),
        ],
        out_specs=[
            pl.BlockSpec((NBLK, PADW), lambda i: (i, 0)),
            pl.BlockSpec((NBLK, BW), lambda i: (i, 0)),
        ],
        out_shape=[
            jax.ShapeDtypeStruct((N_NODES, PADW), jnp.float32),
            jax.ShapeDtypeStruct((N_NODES, BW), jnp.float32),
        ],
    )(h, p_pad, c["wq_p"], c["bq_p"], c["wb_p"], c["bb_p"], c["sa"], c["sb"])


def _phase3(ar, bc, c):
    eblocks = N_EDGES // EBLK
    return pl.pallas_call(
        _edge_body,
        grid=(eblocks,),
        in_specs=[
            pl.BlockSpec((EBLK, PADW), lambda i: (i, 0)),
            pl.BlockSpec((EBLK, BW), lambda i: (i, 0)),
            _FULL(EDGE_DIM, PADW), _FULL(EDGE_DIM, PADW),
            _FULL(EDGE_DIM, EDGE_DIM), _FULL(1, EDGE_DIM),
            _FULL(PADW, PDIM), _FULL(PDIM, PADW),
            _FULL(PDIM, PADW), _FULL(PDIM, PADW), _FULL(1, PADW),
        ],
        out_specs=[
            pl.BlockSpec((EBLK, PADW), lambda i: (i, 0)),
            pl.BlockSpec((EBLK, PADW), lambda i: (i, 0)),
        ],
        out_shape=[
            jax.ShapeDtypeStruct((N_EDGES, PADW), jnp.float32),
            jax.ShapeDtypeStruct((N_EDGES, PADW), jnp.float32),
        ],
    )(ar, bc, c["wke_p"], c["wve_p"], c["We"], c["be_p"], c["shead"],
      c["ehead"], c["f0"], c["f1"], c["pm"])


def _phase5(h, acc, c):
    nblocks = N_NODES // NBLK
    return pl.pallas_call(
        _final_body,
        grid=(nblocks,),
        in_specs=[
            pl.BlockSpec((NBLK, C_Z), lambda i: (i, 0)),
            pl.BlockSpec((NBLK, PADW), lambda i: (i, 0)),
            pl.BlockSpec((NBLK, PADW), lambda i: (i, 0)),
            _FULL(PADW, C_Z), _FULL(1, C_Z),
            _FULL(PADW, PDIM), _FULL(PADW, PDIM),
            _FULL(PDIM, PADW), _FULL(1, PADW),
        ],
        out_specs=pl.BlockSpec((NBLK, C_Z), lambda i: (i, 0)),
        out_shape=jax.ShapeDtypeStruct((N_NODES, C_Z), jnp.float32),
    )(h, acc[0], acc[1], c["wo_p"], c["bo_p"], c["t0"], c["t1"],
      c["ehead"], c["em"])


def kernel(pos, h, edge_index, We, be, Wq, bq, Wk, bk, Wv, bv, Wo, bo):
    f32 = jnp.float32
    row = edge_index[0].astype(jnp.int32)
    col = edge_index[1].astype(jnp.int32)
    c = _build_constants(We, be, Wq, bq, Wk, bk, Wv, bv, Wo, bo)
    p_pad = jnp.zeros((N_NODES, PDIM), f32).at[:, :3].set(pos)

    a_tab, b_tab = _phase1(h, p_pad, c)

    mesh = plsc.VectorSubcoreMesh(core_axis_name="c", subcore_axis_name="s")
    gather = pl.kernel(
        _sc_gather_body,
        out_type=[
            jax.ShapeDtypeStruct((N_EDGES, PADW), f32),
            jax.ShapeDtypeStruct((N_EDGES, BW), f32),
        ],
        mesh=mesh,
        scratch_types=[
            pltpu.VMEM((CG,), jnp.int32),
            pltpu.VMEM((CG,), jnp.int32),
            pltpu.VMEM((CG, PADW), f32),
            pltpu.VMEM((CG, BW), f32),
            pltpu.SemaphoreType.DMA,
        ],
    )
    ar, bc = gather(a_tab, b_tab, row, col)

    o0, o1 = _phase3(ar, bc, c)

    zn = jnp.zeros((NACC, PADW), f32)
    scatter = pl.kernel(
        _sc_scatter_body,
        out_type=jax.ShapeDtypeStruct((SC_CORES, NACC, PADW), f32),
        mesh=mesh,
        scratch_types=[
            pltpu.VMEM((CS,), jnp.int32),
            pltpu.VMEM((CS, PADW), f32),
            pltpu.VMEM_SHARED((NACC, PADW), f32),
            pltpu.SemaphoreType.DMA,
        ],
    )
    acc = scatter(o0, o1, row, zn)

    return _phase5(h, acc, c)


# speedup vs baseline: 46.0109x; 1.1452x over previous
import math

import jax
import jax.numpy as jnp
import numpy as np
from jax import lax
from jax.experimental import pallas as pl
from jax.experimental.pallas import tpu as pltpu
from jax.experimental.pallas import tpu_sc as plsc

C_Z = 128
NUM_HEADS = 12
HEAD_DIM = C_Z // NUM_HEADS
EDGE_DIM = C_Z // 4
N_NODES = 10000
N_EDGES = 320000
NUM_GAUSS = EDGE_DIM
STOP = 15.0
HD = NUM_HEADS * HEAD_DIM
PADW = 128
BW = 2 * PADW
PDIM = 16

SC_CORES = 2
SC_TILES = 16
N_WORKERS = SC_CORES * SC_TILES
K_CHUNKS = 5
ECHUNK = N_EDGES // K_CHUNKS
EPW = ECHUNK // N_WORKERS
CG = 200
NCG = EPW // CG
EPT = ECHUNK // SC_TILES
CS = 200
NCS = EPT // CS
NACC = 10240
NPT = NACC // SC_TILES

_RBF_STEP = STOP / (NUM_GAUSS - 1)
_RBF_COEFF = -0.5 / _RBF_STEP**2
_INV_SQRT_D = 1.0 / math.sqrt(float(HEAD_DIM))

NBLK = 1000
EBLK = 2000


def _prep_body(h_ref, p_ref, wq_ref, bq_ref, wb_ref, bb_ref, sa_ref, sb_ref,
               a_ref, b_ref):
    h = h_ref[...]
    p16 = p_ref[...]
    a_ref[...] = (
        jnp.dot(h, wq_ref[...], preferred_element_type=jnp.float32)
        + bq_ref[...]
        + jnp.dot(p16, sa_ref[...], preferred_element_type=jnp.float32)
    )
    b_ref[...] = (
        jnp.dot(h, wb_ref[...], preferred_element_type=jnp.float32)
        + bb_ref[...]
        + jnp.dot(p16, sb_ref[...], preferred_element_type=jnp.float32)
    )


def _edge_body(ar_ref, bc_ref, wke_ref, wve_ref, we_ref, be_ref, shead_ref,
               ehead_ref, f0_ref, f1_ref, pm_ref, o0_ref, o1_ref):
    ar = ar_ref[...]
    bc1 = bc_ref[:, :PADW]
    bc2 = bc_ref[:, PADW:]
    dd = (ar - bc1) * pm_ref[...]
    d2 = jnp.sum(dd * dd, axis=1, keepdims=True)
    dist = jnp.sqrt(d2 + 1e-12)
    offs = (lax.broadcasted_iota(jnp.int32, (1, NUM_GAUSS), 1)
            .astype(jnp.float32) * _RBF_STEP)
    rbf = jnp.exp(_RBF_COEFF * (dist - offs) ** 2)
    ea = (jnp.dot(rbf, we_ref[...], preferred_element_type=jnp.float32)
          + be_ref[...])
    k = bc1 + jnp.dot(ea, wke_ref[...], preferred_element_type=jnp.float32)
    v = bc2 + jnp.dot(ea, wve_ref[...], preferred_element_type=jnp.float32)
    alpha = jnp.dot(ar * k, shead_ref[...],
                    preferred_element_type=jnp.float32)
    e = jnp.exp(alpha)
    ev = v * jnp.dot(e, ehead_ref[...], preferred_element_type=jnp.float32)
    o0_ref[...] = ev + jnp.dot(e, f0_ref[...],
                               preferred_element_type=jnp.float32)
    o1_ref[...] = jnp.dot(e, f1_ref[...], preferred_element_type=jnp.float32)


def _final_body(h_ref, a0_ref, a1_ref, wo_ref, bo_ref, t0_ref, t1_ref,
                ehead_ref, em_ref, out_ref):
    a0 = a0_ref[...]
    a1 = a1_ref[...]
    num = a0 * em_ref[...]
    den16 = (jnp.dot(a0, t0_ref[...], preferred_element_type=jnp.float32)
             + jnp.dot(a1, t1_ref[...], preferred_element_type=jnp.float32))
    dexp = jnp.dot(den16, ehead_ref[...], preferred_element_type=jnp.float32)
    agg = num / (dexp + 1e-16)
    out_ref[...] = (
        h_ref[...]
        + jnp.dot(agg, wo_ref[...], preferred_element_type=jnp.float32)
        + bo_ref[...]
    )


def _sc_gather_body(a_hbm, b_hbm, row_hbm, col_hbm, ar_hbm, bc_hbm,
                    ri_v, ci_v, a_v, b_v, sem):
    wid = lax.axis_index("s") * SC_CORES + lax.axis_index("c")
    base = wid * EPW

    @pl.loop(0, NCG)
    def _(j):
        off = base + j * CG
        pltpu.sync_copy(row_hbm.at[pl.ds(off, CG)], ri_v)
        pltpu.sync_copy(col_hbm.at[pl.ds(off, CG)], ci_v)
        c1 = pltpu.async_copy(a_hbm.at[ri_v], a_v, sem)
        c2 = pltpu.async_copy(b_hbm.at[ci_v], b_v, sem)
        c1.wait()
        c2.wait()
        pltpu.sync_copy(a_v, ar_hbm.at[pl.ds(off, CG)])
        pltpu.sync_copy(b_v, bc_hbm.at[pl.ds(off, CG)])


def _sc_scatter_body(o0_hbm, o1_hbm, row_hbm, accin_hbm, acc_hbm,
                     ri_v, val_v, acc_s, sem):
    cid = lax.axis_index("c")
    sid = lax.axis_index("s")
    pltpu.sync_copy(accin_hbm.at[cid, pl.ds(sid * NPT, NPT)],
                    acc_s.at[pl.ds(sid * NPT, NPT)])
    plsc.subcore_barrier()

    base = sid * EPT

    def _run(src_hbm):
        @pl.loop(0, NCS)
        def _(j):
            off = base + j * CS
            pltpu.sync_copy(row_hbm.at[pl.ds(off, CS)], ri_v)
            pltpu.sync_copy(src_hbm.at[pl.ds(off, CS)], val_v)
            pltpu.sync_copy(val_v, acc_s.at[ri_v], add=True)

    @pl.when(cid == 0)
    def _():
        _run(o0_hbm)

    @pl.when(cid == 1)
    def _():
        _run(o1_hbm)

    plsc.subcore_barrier()
    pltpu.sync_copy(acc_s.at[pl.ds(sid * NPT, NPT)],
                    acc_hbm.at[cid, pl.ds(sid * NPT, NPT)])


def _build_constants(We, be, Wq, bq, Wk, bk, Wv, bv, Wo, bo):
    f32 = jnp.float32
    s = _INV_SQRT_D
    wq_p = jnp.zeros((C_Z, PADW), f32).at[:, :HD].set(Wq * s)
    bq_p = jnp.zeros((1, PADW), f32).at[0, :HD].set(bq * s)
    wb_p = (jnp.zeros((C_Z, BW), f32)
            .at[:, :HD].set(Wk[:C_Z])
            .at[:, PADW:PADW + HD].set(Wv[:C_Z]))
    bb_p = (jnp.zeros((1, BW), f32)
            .at[0, :HD].set(bk)
            .at[0, PADW:PADW + HD].set(bv))
    wke_p = jnp.zeros((EDGE_DIM, PADW), f32).at[:, :HD].set(Wk[C_Z:])
    wve_p = jnp.zeros((EDGE_DIM, PADW), f32).at[:, :HD].set(Wv[C_Z:])
    wo_p = jnp.zeros((PADW, C_Z), f32).at[:HD, :].set(Wo)
    bo_p = bo.reshape(1, C_Z)
    be_p = be.reshape(1, EDGE_DIM)

    sh = np.zeros((PADW, PDIM), np.float32)
    for dd in range(HD):
        sh[dd, dd // HEAD_DIM] = 1.0
    f0 = np.zeros((PDIM, PADW), np.float32)
    for j in range(8):
        f0[j, HD + j] = 1.0
    f1 = np.zeros((PDIM, PADW), np.float32)
    for j in range(8, NUM_HEADS):
        f1[j, j - 8] = 1.0
    sa = np.zeros((PDIM, PADW), np.float32)
    sb = np.zeros((PDIM, BW), np.float32)
    for j in range(3):
        sa[j, HD + j] = 1.0
        sb[j, HD + j] = 1.0
    pm = np.zeros((1, PADW), np.float32)
    pm[0, HD:HD + 3] = 1.0
    em = np.zeros((1, PADW), np.float32)
    em[0, :HD] = 1.0

    return dict(
        wq_p=wq_p, bq_p=bq_p, wb_p=wb_p, bb_p=bb_p, wke_p=wke_p, wve_p=wve_p,
        wo_p=wo_p, bo_p=bo_p, be_p=be_p, We=We,
        shead=jnp.asarray(sh), ehead=jnp.asarray(np.ascontiguousarray(sh.T)),
        f0=jnp.asarray(f0), f1=jnp.asarray(f1),
        t0=jnp.asarray(np.ascontiguousarray(f0.T)),
        t1=jnp.asarray(np.ascontiguousarray(f1.T)),
        sa=jnp.asarray(sa), sb=jnp.asarray(sb),
        pm=jnp.asarray(pm), em=jnp.asarray(em),
    )


_FULL = lambda r, c: pl.BlockSpec((r, c), lambda i: (0, 0))


def _phase1(h, p_pad, c):
    nblocks = N_NODES // NBLK
    return pl.pallas_call(
        _prep_body,
        grid=(nblocks,),
        in_specs=[
            pl.BlockSpec((NBLK, C_Z), lambda i: (i, 0)),
            pl.BlockSpec((NBLK, PDIM), lambda i: (i, 0)),
            _FULL(C_Z, PADW), _FULL(1, PADW),
            _FULL(C_Z, BW), _FULL(1, BW),
            _FULL(PDIM, PADW), _FULL(PDIM, BW),
        ],
        out_specs=[
            pl.BlockSpec((NBLK, PADW), lambda i: (i, 0)),
            pl.BlockSpec((NBLK, BW), lambda i: (i, 0)),
        ],
        out_shape=[
            jax.ShapeDtypeStruct((N_NODES, PADW), jnp.float32),
            jax.ShapeDtypeStruct((N_NODES, BW), jnp.float32),
        ],
    )(h, p_pad, c["wq_p"], c["bq_p"], c["wb_p"], c["bb_p"], c["sa"], c["sb"])


def _phase3(ar, bc, c):
    eblocks = ECHUNK // EBLK
    return pl.pallas_call(
        _edge_body,
        grid=(eblocks,),
        in_specs=[
            pl.BlockSpec((EBLK, PADW), lambda i: (i, 0)),
            pl.BlockSpec((EBLK, BW), lambda i: (i, 0)),
            _FULL(EDGE_DIM, PADW), _FULL(EDGE_DIM, PADW),
            _FULL(EDGE_DIM, EDGE_DIM), _FULL(1, EDGE_DIM),
            _FULL(PADW, PDIM), _FULL(PDIM, PADW),
            _FULL(PDIM, PADW), _FULL(PDIM, PADW), _FULL(1, PADW),
        ],
        out_specs=[
            pl.BlockSpec((EBLK, PADW), lambda i: (i, 0)),
            pl.BlockSpec((EBLK, PADW), lambda i: (i, 0)),
        ],
        out_shape=[
            jax.ShapeDtypeStruct((ECHUNK, PADW), jnp.float32),
            jax.ShapeDtypeStruct((ECHUNK, PADW), jnp.float32),
        ],
    )(ar, bc, c["wke_p"], c["wve_p"], c["We"], c["be_p"], c["shead"],
      c["ehead"], c["f0"], c["f1"], c["pm"])


def _phase5(h, acc, c):
    nblocks = N_NODES // NBLK
    return pl.pallas_call(
        _final_body,
        grid=(nblocks,),
        in_specs=[
            pl.BlockSpec((NBLK, C_Z), lambda i: (i, 0)),
            pl.BlockSpec((NBLK, PADW), lambda i: (i, 0)),
            pl.BlockSpec((NBLK, PADW), lambda i: (i, 0)),
            _FULL(PADW, C_Z), _FULL(1, C_Z),
            _FULL(PADW, PDIM), _FULL(PADW, PDIM),
            _FULL(PDIM, PADW), _FULL(1, PADW),
        ],
        out_specs=pl.BlockSpec((NBLK, C_Z), lambda i: (i, 0)),
        out_shape=jax.ShapeDtypeStruct((N_NODES, C_Z), jnp.float32),
    )(h, acc[0], acc[1], c["wo_p"], c["bo_p"], c["t0"], c["t1"],
      c["ehead"], c["em"])


def kernel(pos, h, edge_index, We, be, Wq, bq, Wk, bk, Wv, bv, Wo, bo):
    f32 = jnp.float32
    row = edge_index[0].astype(jnp.int32)
    col = edge_index[1].astype(jnp.int32)
    c = _build_constants(We, be, Wq, bq, Wk, bk, Wv, bv, Wo, bo)
    p_pad = jnp.zeros((N_NODES, PDIM), f32).at[:, :3].set(pos)

    a_tab, b_tab = _phase1(h, p_pad, c)

    mesh = plsc.VectorSubcoreMesh(core_axis_name="c", subcore_axis_name="s")
    gather = pl.kernel(
        _sc_gather_body,
        out_type=[
            jax.ShapeDtypeStruct((ECHUNK, PADW), f32),
            jax.ShapeDtypeStruct((ECHUNK, BW), f32),
        ],
        mesh=mesh,
        scratch_types=[
            pltpu.VMEM((CG,), jnp.int32),
            pltpu.VMEM((CG,), jnp.int32),
            pltpu.VMEM((CG, PADW), f32),
            pltpu.VMEM((CG, BW), f32),
            pltpu.SemaphoreType.DMA,
        ],
    )
    scatter = pl.kernel(
        _sc_scatter_body,
        out_type=jax.ShapeDtypeStruct((SC_CORES, NACC, PADW), f32),
        mesh=mesh,
        scratch_types=[
            pltpu.VMEM((CS,), jnp.int32),
            pltpu.VMEM((CS, PADW), f32),
            pltpu.VMEM_SHARED((NACC, PADW), f32),
            pltpu.SemaphoreType.DMA,
        ],
    )

    acc = jnp.zeros((SC_CORES, NACC, PADW), f32)
    for ci in range(K_CHUNKS):
        rc = lax.slice_in_dim(row, ci * ECHUNK, (ci + 1) * ECHUNK)
        cc = lax.slice_in_dim(col, ci * ECHUNK, (ci + 1) * ECHUNK)
        ar, bc = gather(a_tab, b_tab, rc, cc)
        o0, o1 = _phase3(ar, bc, c)
        acc = scatter(o0, o1, rc, acc)

    return _phase5(h, acc, c)


# speedup vs baseline: 47.0347x; 1.0223x over previous
import math

import jax
import jax.numpy as jnp
import numpy as np
from jax import lax
from jax.experimental import pallas as pl
from jax.experimental.pallas import tpu as pltpu
from jax.experimental.pallas import tpu_sc as plsc

C_Z = 128
NUM_HEADS = 12
HEAD_DIM = C_Z // NUM_HEADS
EDGE_DIM = C_Z // 4
N_NODES = 10000
N_EDGES = 320000
NUM_GAUSS = EDGE_DIM
STOP = 15.0
HD = NUM_HEADS * HEAD_DIM
PADW = 128
BW = 2 * PADW
PDIM = 16

SC_CORES = 2
SC_TILES = 16
N_WORKERS = SC_CORES * SC_TILES
K_CHUNKS = 5
ECHUNK = N_EDGES // K_CHUNKS
EPW = ECHUNK // N_WORKERS
CG = 40
NCG = EPW // CG
EPT = ECHUNK // SC_TILES
CS = 160
NCS = EPT // CS
NACC = 10240
NPT = NACC // SC_TILES

_RBF_STEP = STOP / (NUM_GAUSS - 1)
_RBF_COEFF = -0.5 / _RBF_STEP**2
_INV_SQRT_D = 1.0 / math.sqrt(float(HEAD_DIM))

NBLK = 1000
EBLK = 2000


def _prep_body(h_ref, p_ref, wq_ref, bq_ref, wb_ref, bb_ref, sa_ref, sb_ref,
               a_ref, b_ref):
    h = h_ref[...]
    p16 = p_ref[...]
    a_ref[...] = (
        jnp.dot(h, wq_ref[...], preferred_element_type=jnp.float32)
        + bq_ref[...]
        + jnp.dot(p16, sa_ref[...], preferred_element_type=jnp.float32)
    )
    b_ref[...] = (
        jnp.dot(h, wb_ref[...], preferred_element_type=jnp.float32)
        + bb_ref[...]
        + jnp.dot(p16, sb_ref[...], preferred_element_type=jnp.float32)
    )


def _edge_body(ar_ref, bc_ref, wke_ref, wve_ref, we_ref, be_ref, shead_ref,
               ehead_ref, f0_ref, f1_ref, pm_ref, o0_ref, o1_ref):
    ar = ar_ref[...]
    bc1 = bc_ref[:, :PADW]
    bc2 = bc_ref[:, PADW:]
    dd = (ar - bc1) * pm_ref[...]
    d2 = jnp.sum(dd * dd, axis=1, keepdims=True)
    dist = jnp.sqrt(d2 + 1e-12)
    offs = (lax.broadcasted_iota(jnp.int32, (1, NUM_GAUSS), 1)
            .astype(jnp.float32) * _RBF_STEP)
    rbf = jnp.exp(_RBF_COEFF * (dist - offs) ** 2)
    ea = (jnp.dot(rbf, we_ref[...], preferred_element_type=jnp.float32)
          + be_ref[...])
    k = bc1 + jnp.dot(ea, wke_ref[...], preferred_element_type=jnp.float32)
    v = bc2 + jnp.dot(ea, wve_ref[...], preferred_element_type=jnp.float32)
    alpha = jnp.dot(ar * k, shead_ref[...],
                    preferred_element_type=jnp.float32)
    e = jnp.exp(alpha)
    ev = v * jnp.dot(e, ehead_ref[...], preferred_element_type=jnp.float32)
    o0_ref[...] = ev + jnp.dot(e, f0_ref[...],
                               preferred_element_type=jnp.float32)
    o1_ref[...] = jnp.dot(e, f1_ref[...], preferred_element_type=jnp.float32)


def _final_body(h_ref, a0_ref, a1_ref, wo_ref, bo_ref, t0_ref, t1_ref,
                ehead_ref, em_ref, out_ref):
    a0 = a0_ref[...]
    a1 = a1_ref[...]
    num = a0 * em_ref[...]
    den16 = (jnp.dot(a0, t0_ref[...], preferred_element_type=jnp.float32)
             + jnp.dot(a1, t1_ref[...], preferred_element_type=jnp.float32))
    dexp = jnp.dot(den16, ehead_ref[...], preferred_element_type=jnp.float32)
    agg = num / (dexp + 1e-16)
    out_ref[...] = (
        h_ref[...]
        + jnp.dot(agg, wo_ref[...], preferred_element_type=jnp.float32)
        + bo_ref[...]
    )


def _sc_gather_body(a_hbm, b_hbm, row_hbm, col_hbm, ar_hbm, bc_hbm,
                    ri0_v, ri1_v, ci0_v, ci1_v, a_v, b_v, a_s,
                    sg0, sg1, ss0, ss1):
    cid = lax.axis_index("c")
    sid = lax.axis_index("s")
    wid = sid * SC_CORES + cid
    base = wid * EPW
    ri = (ri0_v, ri1_v)
    ci = (ci0_v, ci1_v)
    sg = (sg0, sg1)
    ss = (ss0, ss1)

    del a_s

    gh = [None, None]
    sh = [None, None]
    for j in range(NCG):
        b = j % 2
        if sh[b] is not None:
            sh[b][0].wait()
            sh[b][1].wait()
            sh[b] = None
        off = base + j * CG
        pltpu.sync_copy(row_hbm.at[pl.ds(off, CG)], ri[b])
        pltpu.sync_copy(col_hbm.at[pl.ds(off, CG)], ci[b])
        g1 = pltpu.async_copy(a_hbm.at[ri[b]], a_v.at[b], sg[b])
        g2 = pltpu.async_copy(b_hbm.at[ci[b]], b_v.at[b], sg[b])
        gh[b] = (g1, g2, off)
        pb = (j - 1) % 2
        if j >= 1:
            g1p, g2p, offp = gh[pb]
            g1p.wait()
            g2p.wait()
            s1 = pltpu.async_copy(a_v.at[pb], ar_hbm.at[pl.ds(offp, CG)],
                                  ss[pb])
            s2 = pltpu.async_copy(b_v.at[pb], bc_hbm.at[pl.ds(offp, CG)],
                                  ss[pb])
            sh[pb] = (s1, s2)
    lb = (NCG - 1) % 2
    g1p, g2p, offp = gh[lb]
    g1p.wait()
    g2p.wait()
    s1 = pltpu.async_copy(a_v.at[lb], ar_hbm.at[pl.ds(offp, CG)], ss[lb])
    s2 = pltpu.async_copy(b_v.at[lb], bc_hbm.at[pl.ds(offp, CG)], ss[lb])
    sh[lb] = (s1, s2)
    for pair in sh:
        if pair is not None:
            pair[0].wait()
            pair[1].wait()


def _sc_scatter_body(o0_hbm, o1_hbm, row_hbm, accin_hbm, acc_hbm,
                     ri0_v, ri1_v, val_v, acc_s, sl0, sl1):
    cid = lax.axis_index("c")
    sid = lax.axis_index("s")
    pltpu.sync_copy(accin_hbm.at[cid, pl.ds(sid * NPT, NPT)],
                    acc_s.at[pl.ds(sid * NPT, NPT)])
    plsc.subcore_barrier()

    base = sid * EPT
    ri = (ri0_v, ri1_v)
    sl = (sl0, sl1)

    def _run(src_hbm):
        def _fire(j, b):
            off = base + j * CS
            l1 = pltpu.async_copy(row_hbm.at[pl.ds(off, CS)], ri[b], sl[b])
            l2 = pltpu.async_copy(src_hbm.at[pl.ds(off, CS)], val_v.at[b],
                                  sl[b])
            return (l1, l2)

        h = _fire(0, 0)
        for j in range(NCS):
            b = j % 2
            hn = _fire(j + 1, (j + 1) % 2) if j + 1 < NCS else None
            h[0].wait()
            h[1].wait()
            pltpu.sync_copy(val_v.at[b], acc_s.at[ri[b]], add=True)
            h = hn

    @pl.when(cid == 0)
    def _():
        _run(o0_hbm)

    @pl.when(cid == 1)
    def _():
        _run(o1_hbm)

    plsc.subcore_barrier()
    pltpu.sync_copy(acc_s.at[pl.ds(sid * NPT, NPT)],
                    acc_hbm.at[cid, pl.ds(sid * NPT, NPT)])


def _build_constants(We, be, Wq, bq, Wk, bk, Wv, bv, Wo, bo):
    f32 = jnp.float32
    s = _INV_SQRT_D
    wq_p = jnp.zeros((C_Z, PADW), f32).at[:, :HD].set(Wq * s)
    bq_p = jnp.zeros((1, PADW), f32).at[0, :HD].set(bq * s)
    wb_p = (jnp.zeros((C_Z, BW), f32)
            .at[:, :HD].set(Wk[:C_Z])
            .at[:, PADW:PADW + HD].set(Wv[:C_Z]))
    bb_p = (jnp.zeros((1, BW), f32)
            .at[0, :HD].set(bk)
            .at[0, PADW:PADW + HD].set(bv))
    wke_p = jnp.zeros((EDGE_DIM, PADW), f32).at[:, :HD].set(Wk[C_Z:])
    wve_p = jnp.zeros((EDGE_DIM, PADW), f32).at[:, :HD].set(Wv[C_Z:])
    wo_p = jnp.zeros((PADW, C_Z), f32).at[:HD, :].set(Wo)
    bo_p = bo.reshape(1, C_Z)
    be_p = be.reshape(1, EDGE_DIM)

    sh = np.zeros((PADW, PDIM), np.float32)
    for dd in range(HD):
        sh[dd, dd // HEAD_DIM] = 1.0
    f0 = np.zeros((PDIM, PADW), np.float32)
    for j in range(8):
        f0[j, HD + j] = 1.0
    f1 = np.zeros((PDIM, PADW), np.float32)
    for j in range(8, NUM_HEADS):
        f1[j, j - 8] = 1.0
    sa = np.zeros((PDIM, PADW), np.float32)
    sb = np.zeros((PDIM, BW), np.float32)
    for j in range(3):
        sa[j, HD + j] = 1.0
        sb[j, HD + j] = 1.0
    pm = np.zeros((1, PADW), np.float32)
    pm[0, HD:HD + 3] = 1.0
    em = np.zeros((1, PADW), np.float32)
    em[0, :HD] = 1.0

    return dict(
        wq_p=wq_p, bq_p=bq_p, wb_p=wb_p, bb_p=bb_p, wke_p=wke_p, wve_p=wve_p,
        wo_p=wo_p, bo_p=bo_p, be_p=be_p, We=We,
        shead=jnp.asarray(sh), ehead=jnp.asarray(np.ascontiguousarray(sh.T)),
        f0=jnp.asarray(f0), f1=jnp.asarray(f1),
        t0=jnp.asarray(np.ascontiguousarray(f0.T)),
        t1=jnp.asarray(np.ascontiguousarray(f1.T)),
        sa=jnp.asarray(sa), sb=jnp.asarray(sb),
        pm=jnp.asarray(pm), em=jnp.asarray(em),
    )


_FULL = lambda r, c: pl.BlockSpec((r, c), lambda i: (0, 0))


def _phase1(h, p_pad, c):
    nblocks = N_NODES // NBLK
    return pl.pallas_call(
        _prep_body,
        grid=(nblocks,),
        in_specs=[
            pl.BlockSpec((NBLK, C_Z), lambda i: (i, 0)),
            pl.BlockSpec((NBLK, PDIM), lambda i: (i, 0)),
            _FULL(C_Z, PADW), _FULL(1, PADW),
            _FULL(C_Z, BW), _FULL(1, BW),
            _FULL(PDIM, PADW), _FULL(PDIM, BW),
        ],
        out_specs=[
            pl.BlockSpec((NBLK, PADW), lambda i: (i, 0)),
            pl.BlockSpec((NBLK, BW), lambda i: (i, 0)),
        ],
        out_shape=[
            jax.ShapeDtypeStruct((NACC, PADW), jnp.float32),
            jax.ShapeDtypeStruct((N_NODES, BW), jnp.float32),
        ],
    )(h, p_pad, c["wq_p"], c["bq_p"], c["wb_p"], c["bb_p"], c["sa"], c["sb"])


def _phase3(ar, bc, c):
    eblocks = ECHUNK // EBLK
    return pl.pallas_call(
        _edge_body,
        grid=(eblocks,),
        in_specs=[
            pl.BlockSpec((EBLK, PADW), lambda i: (i, 0)),
            pl.BlockSpec((EBLK, BW), lambda i: (i, 0)),
            _FULL(EDGE_DIM, PADW), _FULL(EDGE_DIM, PADW),
            _FULL(EDGE_DIM, EDGE_DIM), _FULL(1, EDGE_DIM),
            _FULL(PADW, PDIM), _FULL(PDIM, PADW),
            _FULL(PDIM, PADW), _FULL(PDIM, PADW), _FULL(1, PADW),
        ],
        out_specs=[
            pl.BlockSpec((EBLK, PADW), lambda i: (i, 0)),
            pl.BlockSpec((EBLK, PADW), lambda i: (i, 0)),
        ],
        out_shape=[
            jax.ShapeDtypeStruct((ECHUNK, PADW), jnp.float32),
            jax.ShapeDtypeStruct((ECHUNK, PADW), jnp.float32),
        ],
    )(ar, bc, c["wke_p"], c["wve_p"], c["We"], c["be_p"], c["shead"],
      c["ehead"], c["f0"], c["f1"], c["pm"])


def _phase5(h, acc, c):
    nblocks = N_NODES // NBLK
    return pl.pallas_call(
        _final_body,
        grid=(nblocks,),
        in_specs=[
            pl.BlockSpec((NBLK, C_Z), lambda i: (i, 0)),
            pl.BlockSpec((NBLK, PADW), lambda i: (i, 0)),
            pl.BlockSpec((NBLK, PADW), lambda i: (i, 0)),
            _FULL(PADW, C_Z), _FULL(1, C_Z),
            _FULL(PADW, PDIM), _FULL(PADW, PDIM),
            _FULL(PDIM, PADW), _FULL(1, PADW),
        ],
        out_specs=pl.BlockSpec((NBLK, C_Z), lambda i: (i, 0)),
        out_shape=jax.ShapeDtypeStruct((N_NODES, C_Z), jnp.float32),
    )(h, acc[0], acc[1], c["wo_p"], c["bo_p"], c["t0"], c["t1"],
      c["ehead"], c["em"])


def kernel(pos, h, edge_index, We, be, Wq, bq, Wk, bk, Wv, bv, Wo, bo):
    f32 = jnp.float32
    row = edge_index[0].astype(jnp.int32)
    col = edge_index[1].astype(jnp.int32)
    c = _build_constants(We, be, Wq, bq, Wk, bk, Wv, bv, Wo, bo)
    p_pad = jnp.zeros((N_NODES, PDIM), f32).at[:, :3].set(pos)

    a_tab, b_tab = _phase1(h, p_pad, c)

    mesh = plsc.VectorSubcoreMesh(core_axis_name="c", subcore_axis_name="s")
    gather = pl.kernel(
        _sc_gather_body,
        out_type=[
            jax.ShapeDtypeStruct((ECHUNK, PADW), f32),
            jax.ShapeDtypeStruct((ECHUNK, BW), f32),
        ],
        mesh=mesh,
        scratch_types=[
            pltpu.VMEM((CG,), jnp.int32),
            pltpu.VMEM((CG,), jnp.int32),
            pltpu.VMEM((CG,), jnp.int32),
            pltpu.VMEM((CG,), jnp.int32),
            pltpu.VMEM((2, CG, PADW), f32),
            pltpu.VMEM((2, CG, BW), f32),
            pltpu.VMEM_SHARED((NACC, PADW), f32),
            pltpu.SemaphoreType.DMA,
            pltpu.SemaphoreType.DMA,
            pltpu.SemaphoreType.DMA,
            pltpu.SemaphoreType.DMA,
        ],
    )
    scatter = pl.kernel(
        _sc_scatter_body,
        out_type=jax.ShapeDtypeStruct((SC_CORES, NACC, PADW), f32),
        mesh=mesh,
        scratch_types=[
            pltpu.VMEM((CS,), jnp.int32),
            pltpu.VMEM((CS,), jnp.int32),
            pltpu.VMEM((2, CS, PADW), f32),
            pltpu.VMEM_SHARED((NACC, PADW), f32),
            pltpu.SemaphoreType.DMA,
            pltpu.SemaphoreType.DMA,
        ],
    )

    acc = jnp.zeros((SC_CORES, NACC, PADW), f32)
    for ci in range(K_CHUNKS):
        rc = lax.slice_in_dim(row, ci * ECHUNK, (ci + 1) * ECHUNK)
        cc = lax.slice_in_dim(col, ci * ECHUNK, (ci + 1) * ECHUNK)
        ar, bc = gather(a_tab, b_tab, rc, cc)
        o0, o1 = _phase3(ar, bc, c)
        acc = scatter(o0, o1, rc, acc)

    return _phase5(h, acc, c)


# speedup vs baseline: 51.3571x; 1.0919x over previous
import math

import jax
import jax.numpy as jnp
import numpy as np
from jax import lax
from jax.experimental import pallas as pl
from jax.experimental.pallas import tpu as pltpu
from jax.experimental.pallas import tpu_sc as plsc

C_Z = 128
NUM_HEADS = 12
HEAD_DIM = C_Z // NUM_HEADS
EDGE_DIM = C_Z // 4
N_NODES = 10000
N_EDGES = 320000
NUM_GAUSS = EDGE_DIM
STOP = 15.0
HD = NUM_HEADS * HEAD_DIM
PADW = 128
BW = 2 * PADW
PDIM = 16

SC_CORES = 2
SC_TILES = 16
N_WORKERS = SC_CORES * SC_TILES
K_CHUNKS = 5
ECHUNK = N_EDGES // K_CHUNKS
EPW = ECHUNK // N_WORKERS
CG = 80
NCG = EPW // CG
EPT = ECHUNK // SC_TILES
CS = 160
NCS = EPT // CS
NACC = 10240
NPT = NACC // SC_TILES

_RBF_STEP = STOP / (NUM_GAUSS - 1)
_RBF_COEFF = -0.5 / _RBF_STEP**2
_INV_SQRT_D = 1.0 / math.sqrt(float(HEAD_DIM))

NBLK = 1000
EBLK = 2000


def _prep_body(h_ref, p_ref, wq_ref, bq_ref, wb_ref, bb_ref, sa_ref, sb_ref,
               a_ref, b_ref):
    h = h_ref[...]
    p16 = p_ref[...]
    a_ref[...] = (
        jnp.dot(h, wq_ref[...], preferred_element_type=jnp.float32)
        + bq_ref[...]
        + jnp.dot(p16, sa_ref[...], preferred_element_type=jnp.float32)
    )
    b_ref[...] = (
        jnp.dot(h, wb_ref[...], preferred_element_type=jnp.float32)
        + bb_ref[...]
        + jnp.dot(p16, sb_ref[...], preferred_element_type=jnp.float32)
    )


def _edge_body(ar_ref, bc_ref, wke_ref, wve_ref, we_ref, be_ref, shead_ref,
               ehead_ref, f0_ref, f1_ref, pm_ref, o0_ref, o1_ref):
    ar = ar_ref[...]
    bc1 = bc_ref[:, :PADW]
    bc2 = bc_ref[:, PADW:]
    dd = (ar - bc1) * pm_ref[...]
    d2 = jnp.sum(dd * dd, axis=1, keepdims=True)
    dist = jnp.sqrt(d2 + 1e-12)
    offs = (lax.broadcasted_iota(jnp.int32, (1, NUM_GAUSS), 1)
            .astype(jnp.float32) * _RBF_STEP)
    rbf = jnp.exp(_RBF_COEFF * (dist - offs) ** 2)
    ea = (jnp.dot(rbf, we_ref[...], preferred_element_type=jnp.float32)
          + be_ref[...])
    k = bc1 + jnp.dot(ea, wke_ref[...], preferred_element_type=jnp.float32)
    v = bc2 + jnp.dot(ea, wve_ref[...], preferred_element_type=jnp.float32)
    alpha = jnp.dot(ar * k, shead_ref[...],
                    preferred_element_type=jnp.float32)
    e = jnp.exp(alpha)
    ev = v * jnp.dot(e, ehead_ref[...], preferred_element_type=jnp.float32)
    o0_ref[...] = ev + jnp.dot(e, f0_ref[...],
                               preferred_element_type=jnp.float32)
    o1_ref[...] = jnp.dot(e, f1_ref[...], preferred_element_type=jnp.float32)


def _final_body(h_ref, a0_ref, a1_ref, wo_ref, bo_ref, t0_ref, t1_ref,
                ehead_ref, em_ref, out_ref):
    a0 = a0_ref[...]
    a1 = a1_ref[...]
    num = a0 * em_ref[...]
    den16 = (jnp.dot(a0, t0_ref[...], preferred_element_type=jnp.float32)
             + jnp.dot(a1, t1_ref[...], preferred_element_type=jnp.float32))
    dexp = jnp.dot(den16, ehead_ref[...], preferred_element_type=jnp.float32)
    agg = num / (dexp + 1e-16)
    out_ref[...] = (
        h_ref[...]
        + jnp.dot(agg, wo_ref[...], preferred_element_type=jnp.float32)
        + bo_ref[...]
    )


def _sc_gather_body(a_hbm, b_hbm, row_hbm, col_hbm, ar_hbm, bc_hbm,
                    ri0_v, ri1_v, ci0_v, ci1_v, a_v, b_v,
                    sg0, sg1, ss0, ss1):
    cid = lax.axis_index("c")
    sid = lax.axis_index("s")
    wid = sid * SC_CORES + cid
    base = wid * EPW
    ri = (ri0_v, ri1_v)
    ci = (ci0_v, ci1_v)
    sg = (sg0, sg1)
    ss = (ss0, ss1)

    gh = [None, None]
    sh = [None, None]
    for j in range(NCG):
        b = j % 2
        if sh[b] is not None:
            sh[b][0].wait()
            sh[b][1].wait()
            sh[b] = None
        off = base + j * CG
        pltpu.sync_copy(row_hbm.at[pl.ds(off, CG)], ri[b])
        pltpu.sync_copy(col_hbm.at[pl.ds(off, CG)], ci[b])
        g1 = pltpu.async_copy(a_hbm.at[ri[b]], a_v.at[b], sg[b])
        g2 = pltpu.async_copy(b_hbm.at[ci[b]], b_v.at[b], sg[b])
        gh[b] = (g1, g2, off)
        pb = (j - 1) % 2
        if j >= 1:
            g1p, g2p, offp = gh[pb]
            g1p.wait()
            g2p.wait()
            s1 = pltpu.async_copy(a_v.at[pb], ar_hbm.at[pl.ds(offp, CG)],
                                  ss[pb])
            s2 = pltpu.async_copy(b_v.at[pb], bc_hbm.at[pl.ds(offp, CG)],
                                  ss[pb])
            sh[pb] = (s1, s2)
    lb = (NCG - 1) % 2
    g1p, g2p, offp = gh[lb]
    g1p.wait()
    g2p.wait()
    s1 = pltpu.async_copy(a_v.at[lb], ar_hbm.at[pl.ds(offp, CG)], ss[lb])
    s2 = pltpu.async_copy(b_v.at[lb], bc_hbm.at[pl.ds(offp, CG)], ss[lb])
    sh[lb] = (s1, s2)
    for pair in sh:
        if pair is not None:
            pair[0].wait()
            pair[1].wait()


def _sc_scatter_body(o0_hbm, o1_hbm, row_hbm, accin_hbm, acc_hbm,
                     ri0_v, ri1_v, val_v, acc_s, sl0, sl1):
    cid = lax.axis_index("c")
    sid = lax.axis_index("s")
    pltpu.sync_copy(accin_hbm.at[cid, pl.ds(sid * NPT, NPT)],
                    acc_s.at[pl.ds(sid * NPT, NPT)])
    plsc.subcore_barrier()

    base = sid * EPT
    ri = (ri0_v, ri1_v)
    sl = (sl0, sl1)

    def _run(src_hbm):
        def _fire(j, b):
            off = base + j * CS
            l1 = pltpu.async_copy(row_hbm.at[pl.ds(off, CS)], ri[b], sl[b])
            l2 = pltpu.async_copy(src_hbm.at[pl.ds(off, CS)], val_v.at[b],
                                  sl[b])
            return (l1, l2)

        h = _fire(0, 0)
        for j in range(NCS):
            b = j % 2
            hn = _fire(j + 1, (j + 1) % 2) if j + 1 < NCS else None
            h[0].wait()
            h[1].wait()
            pltpu.sync_copy(val_v.at[b], acc_s.at[ri[b]], add=True)
            h = hn

    @pl.when(cid == 0)
    def _():
        _run(o0_hbm)

    @pl.when(cid == 1)
    def _():
        _run(o1_hbm)

    plsc.subcore_barrier()
    pltpu.sync_copy(acc_s.at[pl.ds(sid * NPT, NPT)],
                    acc_hbm.at[cid, pl.ds(sid * NPT, NPT)])


def _build_constants(We, be, Wq, bq, Wk, bk, Wv, bv, Wo, bo):
    f32 = jnp.float32
    s = _INV_SQRT_D
    wq_p = jnp.zeros((C_Z, PADW), f32).at[:, :HD].set(Wq * s)
    bq_p = jnp.zeros((1, PADW), f32).at[0, :HD].set(bq * s)
    wb_p = (jnp.zeros((C_Z, BW), f32)
            .at[:, :HD].set(Wk[:C_Z])
            .at[:, PADW:PADW + HD].set(Wv[:C_Z]))
    bb_p = (jnp.zeros((1, BW), f32)
            .at[0, :HD].set(bk)
            .at[0, PADW:PADW + HD].set(bv))
    wke_p = jnp.zeros((EDGE_DIM, PADW), f32).at[:, :HD].set(Wk[C_Z:])
    wve_p = jnp.zeros((EDGE_DIM, PADW), f32).at[:, :HD].set(Wv[C_Z:])
    wo_p = jnp.zeros((PADW, C_Z), f32).at[:HD, :].set(Wo)
    bo_p = bo.reshape(1, C_Z)
    be_p = be.reshape(1, EDGE_DIM)

    sh = np.zeros((PADW, PDIM), np.float32)
    for dd in range(HD):
        sh[dd, dd // HEAD_DIM] = 1.0
    f0 = np.zeros((PDIM, PADW), np.float32)
    for j in range(8):
        f0[j, HD + j] = 1.0
    f1 = np.zeros((PDIM, PADW), np.float32)
    for j in range(8, NUM_HEADS):
        f1[j, j - 8] = 1.0
    sa = np.zeros((PDIM, PADW), np.float32)
    sb = np.zeros((PDIM, BW), np.float32)
    for j in range(3):
        sa[j, HD + j] = 1.0
        sb[j, HD + j] = 1.0
    pm = np.zeros((1, PADW), np.float32)
    pm[0, HD:HD + 3] = 1.0
    em = np.zeros((1, PADW), np.float32)
    em[0, :HD] = 1.0

    return dict(
        wq_p=wq_p, bq_p=bq_p, wb_p=wb_p, bb_p=bb_p, wke_p=wke_p, wve_p=wve_p,
        wo_p=wo_p, bo_p=bo_p, be_p=be_p, We=We,
        shead=jnp.asarray(sh), ehead=jnp.asarray(np.ascontiguousarray(sh.T)),
        f0=jnp.asarray(f0), f1=jnp.asarray(f1),
        t0=jnp.asarray(np.ascontiguousarray(f0.T)),
        t1=jnp.asarray(np.ascontiguousarray(f1.T)),
        sa=jnp.asarray(sa), sb=jnp.asarray(sb),
        pm=jnp.asarray(pm), em=jnp.asarray(em),
    )


_FULL = lambda r, c: pl.BlockSpec((r, c), lambda i: (0, 0))


def _phase1(h, p_pad, c):
    nblocks = N_NODES // NBLK
    return pl.pallas_call(
        _prep_body,
        grid=(nblocks,),
        in_specs=[
            pl.BlockSpec((NBLK, C_Z), lambda i: (i, 0)),
            pl.BlockSpec((NBLK, PDIM), lambda i: (i, 0)),
            _FULL(C_Z, PADW), _FULL(1, PADW),
            _FULL(C_Z, BW), _FULL(1, BW),
            _FULL(PDIM, PADW), _FULL(PDIM, BW),
        ],
        out_specs=[
            pl.BlockSpec((NBLK, PADW), lambda i: (i, 0)),
            pl.BlockSpec((NBLK, BW), lambda i: (i, 0)),
        ],
        out_shape=[
            jax.ShapeDtypeStruct((NACC, PADW), jnp.float32),
            jax.ShapeDtypeStruct((N_NODES, BW), jnp.float32),
        ],
    )(h, p_pad, c["wq_p"], c["bq_p"], c["wb_p"], c["bb_p"], c["sa"], c["sb"])


def _phase3(ar, bc, c):
    eblocks = ECHUNK // EBLK
    return pl.pallas_call(
        _edge_body,
        grid=(eblocks,),
        in_specs=[
            pl.BlockSpec((EBLK, PADW), lambda i: (i, 0)),
            pl.BlockSpec((EBLK, BW), lambda i: (i, 0)),
            _FULL(EDGE_DIM, PADW), _FULL(EDGE_DIM, PADW),
            _FULL(EDGE_DIM, EDGE_DIM), _FULL(1, EDGE_DIM),
            _FULL(PADW, PDIM), _FULL(PDIM, PADW),
            _FULL(PDIM, PADW), _FULL(PDIM, PADW), _FULL(1, PADW),
        ],
        out_specs=[
            pl.BlockSpec((EBLK, PADW), lambda i: (i, 0)),
            pl.BlockSpec((EBLK, PADW), lambda i: (i, 0)),
        ],
        out_shape=[
            jax.ShapeDtypeStruct((ECHUNK, PADW), jnp.float32),
            jax.ShapeDtypeStruct((ECHUNK, PADW), jnp.float32),
        ],
    )(ar, bc, c["wke_p"], c["wve_p"], c["We"], c["be_p"], c["shead"],
      c["ehead"], c["f0"], c["f1"], c["pm"])


def _phase5(h, acc, c):
    nblocks = N_NODES // NBLK
    return pl.pallas_call(
        _final_body,
        grid=(nblocks,),
        in_specs=[
            pl.BlockSpec((NBLK, C_Z), lambda i: (i, 0)),
            pl.BlockSpec((NBLK, PADW), lambda i: (i, 0)),
            pl.BlockSpec((NBLK, PADW), lambda i: (i, 0)),
            _FULL(PADW, C_Z), _FULL(1, C_Z),
            _FULL(PADW, PDIM), _FULL(PADW, PDIM),
            _FULL(PDIM, PADW), _FULL(1, PADW),
        ],
        out_specs=pl.BlockSpec((NBLK, C_Z), lambda i: (i, 0)),
        out_shape=jax.ShapeDtypeStruct((N_NODES, C_Z), jnp.float32),
    )(h, acc[0], acc[1], c["wo_p"], c["bo_p"], c["t0"], c["t1"],
      c["ehead"], c["em"])


def kernel(pos, h, edge_index, We, be, Wq, bq, Wk, bk, Wv, bv, Wo, bo):
    f32 = jnp.float32
    row = edge_index[0].astype(jnp.int32)
    col = edge_index[1].astype(jnp.int32)
    c = _build_constants(We, be, Wq, bq, Wk, bk, Wv, bv, Wo, bo)
    p_pad = jnp.zeros((N_NODES, PDIM), f32).at[:, :3].set(pos)

    a_tab, b_tab = _phase1(h, p_pad, c)

    mesh = plsc.VectorSubcoreMesh(core_axis_name="c", subcore_axis_name="s")
    gather = pl.kernel(
        _sc_gather_body,
        out_type=[
            jax.ShapeDtypeStruct((ECHUNK, PADW), f32),
            jax.ShapeDtypeStruct((ECHUNK, BW), f32),
        ],
        mesh=mesh,
        scratch_types=[
            pltpu.VMEM((CG,), jnp.int32),
            pltpu.VMEM((CG,), jnp.int32),
            pltpu.VMEM((CG,), jnp.int32),
            pltpu.VMEM((CG,), jnp.int32),
            pltpu.VMEM((2, CG, PADW), f32),
            pltpu.VMEM((2, CG, BW), f32),
            pltpu.SemaphoreType.DMA,
            pltpu.SemaphoreType.DMA,
            pltpu.SemaphoreType.DMA,
            pltpu.SemaphoreType.DMA,
        ],
    )
    scatter = pl.kernel(
        _sc_scatter_body,
        out_type=jax.ShapeDtypeStruct((SC_CORES, NACC, PADW), f32),
        mesh=mesh,
        scratch_types=[
            pltpu.VMEM((CS,), jnp.int32),
            pltpu.VMEM((CS,), jnp.int32),
            pltpu.VMEM((2, CS, PADW), f32),
            pltpu.VMEM_SHARED((NACC, PADW), f32),
            pltpu.SemaphoreType.DMA,
            pltpu.SemaphoreType.DMA,
        ],
    )

    acc = jnp.zeros((SC_CORES, NACC, PADW), f32)
    for ci in range(K_CHUNKS):
        rc = lax.slice_in_dim(row, ci * ECHUNK, (ci + 1) * ECHUNK)
        cc = lax.slice_in_dim(col, ci * ECHUNK, (ci + 1) * ECHUNK)
        ar, bc = gather(a_tab, b_tab, rc, cc)
        o0, o1 = _phase3(ar, bc, c)
        acc = scatter(o0, o1, rc, acc)

    return _phase5(h, acc, c)


# speedup vs baseline: 53.7724x; 1.0470x over previous
import math

import jax
import jax.numpy as jnp
import numpy as np
from jax import lax
from jax.experimental import pallas as pl
from jax.experimental.pallas import tpu as pltpu
from jax.experimental.pallas import tpu_sc as plsc

C_Z = 128
NUM_HEADS = 12
HEAD_DIM = C_Z // NUM_HEADS
EDGE_DIM = C_Z // 4
N_NODES = 10000
N_EDGES = 320000
NUM_GAUSS = EDGE_DIM
STOP = 15.0
HD = NUM_HEADS * HEAD_DIM
PADW = 128
BW = 2 * PADW
PDIM = 16

SC_CORES = 2
SC_TILES = 16
N_WORKERS = SC_CORES * SC_TILES
K_CHUNKS = 5
ECHUNK = N_EDGES // K_CHUNKS
EPW = ECHUNK // N_WORKERS
CG = 80
NCG = EPW // CG
EPT = ECHUNK // SC_TILES
CS = 160
NCS = EPT // CS
NACC = 10240
NPT = NACC // SC_TILES

_RBF_STEP = STOP / (NUM_GAUSS - 1)
_RBF_COEFF = -0.5 / _RBF_STEP**2
_INV_SQRT_D = 1.0 / math.sqrt(float(HEAD_DIM))

NBLK = 1000
EBLK = 2000


def _prep_body(h_ref, p_ref, wq_ref, bq_ref, wb_ref, bb_ref, sa_ref, sb_ref,
               a_ref, b_ref):
    h = h_ref[...]
    p16 = p_ref[...]
    a_ref[...] = (
        jnp.dot(h, wq_ref[...], preferred_element_type=jnp.float32)
        + bq_ref[...]
        + jnp.dot(p16, sa_ref[...], preferred_element_type=jnp.float32)
    )
    b_ref[...] = (
        jnp.dot(h, wb_ref[...], preferred_element_type=jnp.float32)
        + bb_ref[...]
        + jnp.dot(p16, sb_ref[...], preferred_element_type=jnp.float32)
    )


def _edge_body(ar_ref, bc_ref, wke_ref, wve_ref, we_ref, be_ref, shead_ref,
               ehead_ref, f0_ref, f1_ref, pm_ref, o0_ref, o1_ref):
    ar = ar_ref[...]
    bc1 = bc_ref[:, :PADW]
    bc2 = bc_ref[:, PADW:]
    dd = (ar - bc1) * pm_ref[...]
    d2 = jnp.sum(dd * dd, axis=1, keepdims=True)
    dist = jnp.sqrt(d2 + 1e-12)
    offs = (lax.broadcasted_iota(jnp.int32, (1, NUM_GAUSS), 1)
            .astype(jnp.float32) * _RBF_STEP)
    rbf = jnp.exp(_RBF_COEFF * (dist - offs) ** 2)
    ea = (jnp.dot(rbf, we_ref[...], preferred_element_type=jnp.float32)
          + be_ref[...])
    k = bc1 + jnp.dot(ea, wke_ref[...], preferred_element_type=jnp.float32)
    v = bc2 + jnp.dot(ea, wve_ref[...], preferred_element_type=jnp.float32)
    alpha = jnp.dot(ar * k, shead_ref[...],
                    preferred_element_type=jnp.float32)
    e = jnp.exp(alpha)
    ev = v * jnp.dot(e, ehead_ref[...], preferred_element_type=jnp.float32)
    o0_ref[...] = ev + jnp.dot(e, f0_ref[...],
                               preferred_element_type=jnp.float32)
    o1_ref[...] = jnp.dot(e, f1_ref[...], preferred_element_type=jnp.float32)


def _final_body(h_ref, a0_ref, a1_ref, wo_ref, bo_ref, t0_ref, t1_ref,
                ehead_ref, em_ref, out_ref):
    a0 = a0_ref[...]
    a1 = a1_ref[...]
    num = a0 * em_ref[...]
    den16 = (jnp.dot(a0, t0_ref[...], preferred_element_type=jnp.float32)
             + jnp.dot(a1, t1_ref[...], preferred_element_type=jnp.float32))
    dexp = jnp.dot(den16, ehead_ref[...], preferred_element_type=jnp.float32)
    agg = num / (dexp + 1e-16)
    out_ref[...] = (
        h_ref[...]
        + jnp.dot(agg, wo_ref[...], preferred_element_type=jnp.float32)
        + bo_ref[...]
    )


def _sc_gather_body(a_hbm, b_hbm, row_hbm, col_hbm, ar_hbm, bc_hbm,
                    ria_v, cia_v, a_v, b_v,
                    sg0, sg1, ss0, ss1):
    cid = lax.axis_index("c")
    sid = lax.axis_index("s")
    wid = sid * SC_CORES + cid
    base = wid * EPW
    sg = (sg0, sg1)
    ss = (ss0, ss1)

    pltpu.sync_copy(row_hbm.at[pl.ds(base, EPW)], ria_v)
    pltpu.sync_copy(col_hbm.at[pl.ds(base, EPW)], cia_v)

    gh = [None, None]
    sh = [None, None]
    for j in range(NCG):
        b = j % 2
        if sh[b] is not None:
            sh[b][0].wait()
            sh[b][1].wait()
            sh[b] = None
        off = base + j * CG
        g1 = pltpu.async_copy(a_hbm.at[ria_v.at[pl.ds(j * CG, CG)]],
                              a_v.at[b], sg[b])
        g2 = pltpu.async_copy(b_hbm.at[cia_v.at[pl.ds(j * CG, CG)]],
                              b_v.at[b], sg[b])
        gh[b] = (g1, g2, off)
        pb = (j - 1) % 2
        if j >= 1:
            g1p, g2p, offp = gh[pb]
            g1p.wait()
            g2p.wait()
            s1 = pltpu.async_copy(a_v.at[pb], ar_hbm.at[pl.ds(offp, CG)],
                                  ss[pb])
            s2 = pltpu.async_copy(b_v.at[pb], bc_hbm.at[pl.ds(offp, CG)],
                                  ss[pb])
            sh[pb] = (s1, s2)
    lb = (NCG - 1) % 2
    g1p, g2p, offp = gh[lb]
    g1p.wait()
    g2p.wait()
    s1 = pltpu.async_copy(a_v.at[lb], ar_hbm.at[pl.ds(offp, CG)], ss[lb])
    s2 = pltpu.async_copy(b_v.at[lb], bc_hbm.at[pl.ds(offp, CG)], ss[lb])
    sh[lb] = (s1, s2)
    for pair in sh:
        if pair is not None:
            pair[0].wait()
            pair[1].wait()


def _sc_scatter_body(o0_hbm, o1_hbm, row_hbm, accin_hbm, acc_hbm,
                     ri0_v, ri1_v, val_v, acc_s, sl0, sl1):
    cid = lax.axis_index("c")
    sid = lax.axis_index("s")
    pltpu.sync_copy(accin_hbm.at[cid, pl.ds(sid * NPT, NPT)],
                    acc_s.at[pl.ds(sid * NPT, NPT)])
    plsc.subcore_barrier()

    base = sid * EPT
    ri = (ri0_v, ri1_v)
    sl = (sl0, sl1)

    def _run(src_hbm):
        def _fire(j, b):
            off = base + j * CS
            l1 = pltpu.async_copy(row_hbm.at[pl.ds(off, CS)], ri[b], sl[b])
            l2 = pltpu.async_copy(src_hbm.at[pl.ds(off, CS)], val_v.at[b],
                                  sl[b])
            return (l1, l2)

        h = _fire(0, 0)
        for j in range(NCS):
            b = j % 2
            hn = _fire(j + 1, (j + 1) % 2) if j + 1 < NCS else None
            h[0].wait()
            h[1].wait()
            pltpu.sync_copy(val_v.at[b], acc_s.at[ri[b]], add=True)
            h = hn

    @pl.when(cid == 0)
    def _():
        _run(o0_hbm)

    @pl.when(cid == 1)
    def _():
        _run(o1_hbm)

    plsc.subcore_barrier()
    pltpu.sync_copy(acc_s.at[pl.ds(sid * NPT, NPT)],
                    acc_hbm.at[cid, pl.ds(sid * NPT, NPT)])


def _build_constants(We, be, Wq, bq, Wk, bk, Wv, bv, Wo, bo):
    f32 = jnp.float32
    s = _INV_SQRT_D
    wq_p = jnp.zeros((C_Z, PADW), f32).at[:, :HD].set(Wq * s)
    bq_p = jnp.zeros((1, PADW), f32).at[0, :HD].set(bq * s)
    wb_p = (jnp.zeros((C_Z, BW), f32)
            .at[:, :HD].set(Wk[:C_Z])
            .at[:, PADW:PADW + HD].set(Wv[:C_Z]))
    bb_p = (jnp.zeros((1, BW), f32)
            .at[0, :HD].set(bk)
            .at[0, PADW:PADW + HD].set(bv))
    wke_p = jnp.zeros((EDGE_DIM, PADW), f32).at[:, :HD].set(Wk[C_Z:])
    wve_p = jnp.zeros((EDGE_DIM, PADW), f32).at[:, :HD].set(Wv[C_Z:])
    wo_p = jnp.zeros((PADW, C_Z), f32).at[:HD, :].set(Wo)
    bo_p = bo.reshape(1, C_Z)
    be_p = be.reshape(1, EDGE_DIM)

    sh = np.zeros((PADW, PDIM), np.float32)
    for dd in range(HD):
        sh[dd, dd // HEAD_DIM] = 1.0
    f0 = np.zeros((PDIM, PADW), np.float32)
    for j in range(8):
        f0[j, HD + j] = 1.0
    f1 = np.zeros((PDIM, PADW), np.float32)
    for j in range(8, NUM_HEADS):
        f1[j, j - 8] = 1.0
    sa = np.zeros((PDIM, PADW), np.float32)
    sb = np.zeros((PDIM, BW), np.float32)
    for j in range(3):
        sa[j, HD + j] = 1.0
        sb[j, HD + j] = 1.0
    pm = np.zeros((1, PADW), np.float32)
    pm[0, HD:HD + 3] = 1.0
    em = np.zeros((1, PADW), np.float32)
    em[0, :HD] = 1.0

    return dict(
        wq_p=wq_p, bq_p=bq_p, wb_p=wb_p, bb_p=bb_p, wke_p=wke_p, wve_p=wve_p,
        wo_p=wo_p, bo_p=bo_p, be_p=be_p, We=We,
        shead=jnp.asarray(sh), ehead=jnp.asarray(np.ascontiguousarray(sh.T)),
        f0=jnp.asarray(f0), f1=jnp.asarray(f1),
        t0=jnp.asarray(np.ascontiguousarray(f0.T)),
        t1=jnp.asarray(np.ascontiguousarray(f1.T)),
        sa=jnp.asarray(sa), sb=jnp.asarray(sb),
        pm=jnp.asarray(pm), em=jnp.asarray(em),
    )


_FULL = lambda r, c: pl.BlockSpec((r, c), lambda i: (0, 0))


def _phase1(h, p_pad, c):
    nblocks = N_NODES // NBLK
    return pl.pallas_call(
        _prep_body,
        grid=(nblocks,),
        in_specs=[
            pl.BlockSpec((NBLK, C_Z), lambda i: (i, 0)),
            pl.BlockSpec((NBLK, PDIM), lambda i: (i, 0)),
            _FULL(C_Z, PADW), _FULL(1, PADW),
            _FULL(C_Z, BW), _FULL(1, BW),
            _FULL(PDIM, PADW), _FULL(PDIM, BW),
        ],
        out_specs=[
            pl.BlockSpec((NBLK, PADW), lambda i: (i, 0)),
            pl.BlockSpec((NBLK, BW), lambda i: (i, 0)),
        ],
        out_shape=[
            jax.ShapeDtypeStruct((NACC, PADW), jnp.float32),
            jax.ShapeDtypeStruct((N_NODES, BW), jnp.float32),
        ],
    )(h, p_pad, c["wq_p"], c["bq_p"], c["wb_p"], c["bb_p"], c["sa"], c["sb"])


def _phase3(ar, bc, c):
    eblocks = ECHUNK // EBLK
    return pl.pallas_call(
        _edge_body,
        grid=(eblocks,),
        in_specs=[
            pl.BlockSpec((EBLK, PADW), lambda i: (i, 0)),
            pl.BlockSpec((EBLK, BW), lambda i: (i, 0)),
            _FULL(EDGE_DIM, PADW), _FULL(EDGE_DIM, PADW),
            _FULL(EDGE_DIM, EDGE_DIM), _FULL(1, EDGE_DIM),
            _FULL(PADW, PDIM), _FULL(PDIM, PADW),
            _FULL(PDIM, PADW), _FULL(PDIM, PADW), _FULL(1, PADW),
        ],
        out_specs=[
            pl.BlockSpec((EBLK, PADW), lambda i: (i, 0)),
            pl.BlockSpec((EBLK, PADW), lambda i: (i, 0)),
        ],
        out_shape=[
            jax.ShapeDtypeStruct((ECHUNK, PADW), jnp.float32),
            jax.ShapeDtypeStruct((ECHUNK, PADW), jnp.float32),
        ],
    )(ar, bc, c["wke_p"], c["wve_p"], c["We"], c["be_p"], c["shead"],
      c["ehead"], c["f0"], c["f1"], c["pm"])


def _phase5(h, acc, c):
    nblocks = N_NODES // NBLK
    return pl.pallas_call(
        _final_body,
        grid=(nblocks,),
        in_specs=[
            pl.BlockSpec((NBLK, C_Z), lambda i: (i, 0)),
            pl.BlockSpec((NBLK, PADW), lambda i: (i, 0)),
            pl.BlockSpec((NBLK, PADW), lambda i: (i, 0)),
            _FULL(PADW, C_Z), _FULL(1, C_Z),
            _FULL(PADW, PDIM), _FULL(PADW, PDIM),
            _FULL(PDIM, PADW), _FULL(1, PADW),
        ],
        out_specs=pl.BlockSpec((NBLK, C_Z), lambda i: (i, 0)),
        out_shape=jax.ShapeDtypeStruct((N_NODES, C_Z), jnp.float32),
    )(h, acc[0], acc[1], c["wo_p"], c["bo_p"], c["t0"], c["t1"],
      c["ehead"], c["em"])


def kernel(pos, h, edge_index, We, be, Wq, bq, Wk, bk, Wv, bv, Wo, bo):
    f32 = jnp.float32
    row = edge_index[0].astype(jnp.int32)
    col = edge_index[1].astype(jnp.int32)
    c = _build_constants(We, be, Wq, bq, Wk, bk, Wv, bv, Wo, bo)
    p_pad = jnp.zeros((N_NODES, PDIM), f32).at[:, :3].set(pos)

    a_tab, b_tab = _phase1(h, p_pad, c)

    mesh = plsc.VectorSubcoreMesh(core_axis_name="c", subcore_axis_name="s")
    gather = pl.kernel(
        _sc_gather_body,
        out_type=[
            jax.ShapeDtypeStruct((ECHUNK, PADW), f32),
            jax.ShapeDtypeStruct((ECHUNK, BW), f32),
        ],
        mesh=mesh,
        scratch_types=[
            pltpu.VMEM((EPW,), jnp.int32),
            pltpu.VMEM((EPW,), jnp.int32),
            pltpu.VMEM((2, CG, PADW), f32),
            pltpu.VMEM((2, CG, BW), f32),
            pltpu.SemaphoreType.DMA,
            pltpu.SemaphoreType.DMA,
            pltpu.SemaphoreType.DMA,
            pltpu.SemaphoreType.DMA,
        ],
    )
    scatter = pl.kernel(
        _sc_scatter_body,
        out_type=jax.ShapeDtypeStruct((SC_CORES, NACC, PADW), f32),
        mesh=mesh,
        scratch_types=[
            pltpu.VMEM((CS,), jnp.int32),
            pltpu.VMEM((CS,), jnp.int32),
            pltpu.VMEM((2, CS, PADW), f32),
            pltpu.VMEM_SHARED((NACC, PADW), f32),
            pltpu.SemaphoreType.DMA,
            pltpu.SemaphoreType.DMA,
        ],
    )

    acc = jnp.zeros((SC_CORES, NACC, PADW), f32)
    for ci in range(K_CHUNKS):
        rc = lax.slice_in_dim(row, ci * ECHUNK, (ci + 1) * ECHUNK)
        cc = lax.slice_in_dim(col, ci * ECHUNK, (ci + 1) * ECHUNK)
        ar, bc = gather(a_tab, b_tab, rc, cc)
        o0, o1 = _phase3(ar, bc, c)
        acc = scatter(o0, o1, rc, acc)

    return _phase5(h, acc, c)
